# baseline (speedup 1.0000x reference)
import functools

import jax
import jax.numpy as jnp
from jax.experimental import pallas as pl
from jax.experimental.pallas import tpu as pltpu

MDET = 25
IOU_T = 0.7
CONF_T = 0.25
OFS_WH = 7680.0
HP = jax.lax.Precision.HIGHEST
F32 = jnp.float32


def _ceil(a, b):
    return -(-a // b) * b


def _red2(v, op):
    t = op(v, axis=0, keepdims=True)
    return op(t, axis=1, keepdims=True)


def _nms_body(x_ref, aw_ref, ah_ref, o_ref, s_ref, *, nx, ny, npts):
    rp = s_ref.shape[0]
    o_ref[...] = jnp.zeros_like(o_ref)
    ni = (jax.lax.broadcasted_iota(jnp.int32, (rp, 128), 0) * 128
          + jax.lax.broadcasted_iota(jnp.int32, (rp, 128), 1))
    gx = (ni % nx).astype(F32)
    gy = ((ni // nx) % ny).astype(F32)

    def sg(k):
        return jax.nn.sigmoid(x_ref[0, k])

    bx = sg(0) * 2.0 + gx - 0.5
    by = sg(1) * 2.0 + gy - 0.5
    bw = (sg(2) * 2.0) ** 2 * aw_ref[...]
    bh = (sg(3) * 2.0) ** 2 * ah_ref[...]
    x1 = bx - bw * 0.5
    y1 = by - bh * 0.5
    x2 = bx + bw * 0.5
    y2 = by + bh * 0.5
    obj = sg(4)
    m = sg(5) * obj
    ci = jnp.zeros((rp, 128), jnp.int32)
    for k in range(6, 85):
        c = sg(k) * obj
        upd = c > m
        m = jnp.where(upd, c, m)
        ci = jnp.where(upd, k - 5, ci)
    valid = (obj > CONF_T) & (m > CONF_T)
    s0 = jnp.where(valid & (ni < npts), m, -1.0)
    ofs = ci.astype(F32) * OFS_WH
    area = (x2 - x1) * (y2 - y1)
    s_ref[...] = s0
    nif = ni.astype(F32)
    lane = jax.lax.broadcasted_iota(jnp.int32, (1, 128), 1)

    def body(t, carry):
        sv = s_ref[...]
        mx = _red2(sv, jnp.max)
        sel = sv >= mx
        ii = _red2(jnp.where(sel, nif, 1e9), jnp.min)
        oh = (nif == ii).astype(F32)

        def pick(v):
            return _red2(oh * v, jnp.sum)

        px1 = pick(x1)
        py1 = pick(y1)
        px2 = pick(x2)
        py2 = pick(y2)
        pofs = pick(ofs)
        parea = pick(area)
        vflag = mx > 0.0
        qx1 = px1 + pofs
        qy1 = py1 + pofs
        qx2 = px2 + pofs
        qy2 = py2 + pofs
        xx1 = jnp.maximum(x1 + ofs, qx1)
        yy1 = jnp.maximum(y1 + ofs, qy1)
        xx2 = jnp.minimum(x2 + ofs, qx2)
        yy2 = jnp.minimum(y2 + ofs, qy2)
        inter = jnp.maximum(xx2 - xx1, 0.0) * jnp.maximum(yy2 - yy1, 0.0)
        iou = inter / (parea + area - inter + 1e-9)
        supp = (iou > IOU_T) | (oh > 0.0)
        s_ref[...] = jnp.where(supp, -1.0, sv)
        row = (jnp.where(lane == 0, jnp.where(vflag, px1, 0.0), 0.0)
               + jnp.where(lane == 1, jnp.where(vflag, py1, 0.0), 0.0)
               + jnp.where(lane == 2, jnp.where(vflag, px2, 0.0), 0.0)
               + jnp.where(lane == 3, jnp.where(vflag, py2, 0.0), 0.0)
               + jnp.where(lane == 4, jnp.where(vflag, 1.0, 0.0), 0.0))
        o_ref[0, pl.ds(t, 1), :] = row
        return carry

    jax.lax.fori_loop(0, MDET, body, 0)


def _nms_call(xt, aw, ah, nx, ny, npts):
    bs, no, rp, _ = xt.shape
    return pl.pallas_call(
        functools.partial(_nms_body, nx=nx, ny=ny, npts=npts),
        grid=(bs,),
        in_specs=[
            pl.BlockSpec((1, no, rp, 128), lambda b: (b, 0, 0, 0)),
            pl.BlockSpec((rp, 128), lambda b: (0, 0)),
            pl.BlockSpec((rp, 128), lambda b: (0, 0)),
        ],
        out_specs=pl.BlockSpec((1, 32, 128), lambda b: (b, 0, 0)),
        out_shape=jax.ShapeDtypeStruct((bs, 32, 128), F32),
        scratch_shapes=[pltpu.VMEM((rp, 128), F32)],
        compiler_params=pltpu.CompilerParams(dimension_semantics=("parallel",)),
    )(xt, aw, ah)


def _tent_w(lo, step, bound, wpad, nrows):
    col = jax.lax.broadcasted_iota(jnp.int32, (nrows, wpad), 1).astype(F32)
    pq = (jax.lax.broadcasted_iota(jnp.int32, (nrows, 1), 0) % 7).astype(F32)
    w = jnp.zeros((nrows, wpad), F32)
    for g in (0, 1):
        s = lo + step * (pq + (g + 0.5) * 0.5)
        inb = (s >= -1.0) & (s <= float(bound))
        scl = jnp.clip(s, 0.0, float(bound - 1))
        w = w + jnp.where(inb, jnp.maximum(0.0, 1.0 - jnp.abs(scl - col)), 0.0)
    return w * 0.5


def _roiy_body(nms_ref, feat_ref, a_ref, wx_ref, *, h, w, hpad, wpad):
    rsel = (jax.lax.broadcasted_iota(jnp.int32, (176, 32), 0) // 7
            == jax.lax.broadcasted_iota(jnp.int32, (176, 32), 1)).astype(F32)
    br = jax.lax.dot(rsel, nms_ref[0], precision=HP)
    x1c = br[:, 0:1]
    y1c = br[:, 1:2]
    x2c = br[:, 2:3]
    y2c = br[:, 3:4]
    bh = jnp.maximum(y2c - y1c, 1.0) / 7.0
    bw = jnp.maximum(x2c - x1c, 1.0) / 7.0
    wy = _tent_w(y1c, bh, h, hpad, 176)
    a_ref[0] = jax.lax.dot(wy, feat_ref[0], precision=HP)
    wx_ref[0] = _tent_w(x1c, bw, w, wpad, 176)


def _roiy_call(nmso, feath, h, w, hpad, wpad):
    bs = feath.shape[0]
    wc = feath.shape[2]
    return pl.pallas_call(
        functools.partial(_roiy_body, h=h, w=w, hpad=hpad, wpad=wpad),
        grid=(bs,),
        in_specs=[
            pl.BlockSpec((1, 32, 128), lambda b: (b, 0, 0)),
            pl.BlockSpec((1, hpad, wc), lambda b: (b, 0, 0)),
        ],
        out_specs=[
            pl.BlockSpec((1, 176, wc), lambda b: (b, 0, 0)),
            pl.BlockSpec((1, 176, wpad), lambda b: (b, 0, 0)),
        ],
        out_shape=[
            jax.ShapeDtypeStruct((bs, 176, wc), F32),
            jax.ShapeDtypeStruct((bs, 176, wpad), F32),
        ],
        compiler_params=pltpu.CompilerParams(dimension_semantics=("parallel",)),
    )(nmso, feath)


def _poolconv_body(a2_ref, wx_ref, cw_ref, cb_ref, o_ref, p_scr, *, wpad, c):
    p_scr[...] = jnp.zeros_like(p_scr)
    for d in range(MDET):
        pd = jax.lax.dot(wx_ref[0, d * 7:(d + 1) * 7, :],
                         a2_ref[0, d * wpad:(d + 1) * wpad, :], precision=HP)
        p_scr[d * 7:(d + 1) * 7, :] = pd
    for py in range(7):
        hp_ = jax.lax.dot(p_scr[:, py * c:(py + 1) * c], cw_ref[...],
                          precision=HP) + cb_ref[...]
        o_ref[0, :, py * 84:(py + 1) * 84] = hp_


def _poolconv_call(a2, wx, cwt, cb, wpad, c):
    bs = a2.shape[0]
    return pl.pallas_call(
        functools.partial(_poolconv_body, wpad=wpad, c=c),
        grid=(bs,),
        in_specs=[
            pl.BlockSpec((1, MDET * wpad, 7 * c), lambda b: (b, 0, 0)),
            pl.BlockSpec((1, 176, wpad), lambda b: (b, 0, 0)),
            pl.BlockSpec((c, 84), lambda b: (0, 0)),
            pl.BlockSpec((1, 84), lambda b: (0, 0)),
        ],
        out_specs=pl.BlockSpec((1, 176, 588), lambda b: (b, 0, 0)),
        out_shape=jax.ShapeDtypeStruct((bs, 176, 588), F32),
        scratch_shapes=[pltpu.VMEM((176, 7 * c), F32)],
        compiler_params=pltpu.CompilerParams(dimension_semantics=("parallel",)),
    )(a2, wx, cwt, cb)


def _fc_body(h_ref, w1_ref, b1_ref, w2_ref, b2_ref, w3_ref, b3_ref, vm_ref,
             o_ref):
    z = jax.nn.relu(jax.lax.dot(h_ref[0], w1_ref[0], precision=HP)
                    + b1_ref[0])
    z = jax.nn.relu(jax.lax.dot(z, w2_ref[0], precision=HP) + b2_ref[0])
    z = jax.lax.dot(z, w3_ref[0], precision=HP) + b3_ref[0]
    o_ref[0] = z * vm_ref[0]


def _fc_call(h4, w1, b1, w2, b2, w3, b3, vm):
    nl = h4.shape[0]
    return pl.pallas_call(
        _fc_body,
        grid=(nl,),
        in_specs=[
            pl.BlockSpec((1, 104, 4120), lambda l: (l, 0, 0)),
            pl.BlockSpec((1, 4120, 104), lambda l: (l, 0, 0)),
            pl.BlockSpec((1, 1, 104), lambda l: (l, 0, 0)),
            pl.BlockSpec((1, 104, 104), lambda l: (l, 0, 0)),
            pl.BlockSpec((1, 1, 104), lambda l: (l, 0, 0)),
            pl.BlockSpec((1, 104, 8), lambda l: (l, 0, 0)),
            pl.BlockSpec((1, 1, 8), lambda l: (l, 0, 0)),
            pl.BlockSpec((1, 104, 8), lambda l: (l, 0, 0)),
        ],
        out_specs=pl.BlockSpec((1, 104, 8), lambda l: (l, 0, 0)),
        out_shape=jax.ShapeDtypeStruct((nl, 104, 8), F32),
        compiler_params=pltpu.CompilerParams(dimension_semantics=("parallel",)),
    )(h4, w1, b1, w2, b2, w3, b3, vm)


def kernel(features_0, features_1, features_2, x_0, x_1, x_2, conv_w0,
           conv_w1, conv_w2, conv_b, fc1_w, fc1_b, fc2_w, fc2_b, fc3_w,
           fc3_b, anchors):
    feats = (features_0, features_1, features_2)
    xs = (x_0, x_1, x_2)
    cws = (conv_w0, conv_w1, conv_w2)
    h4s, vms = [], []
    for i in range(3):
        feat, xl, cw = feats[i], xs[i], cws[i]
        bs, c, h, w = feat.shape
        _, na, ny, nx, no = xl.shape
        n = na * ny * nx
        rp = _ceil(-(-n // 128), 8)
        npad = rp * 128
        xt = xl.transpose(0, 4, 1, 2, 3).reshape(bs, no, n)
        xt = jnp.pad(xt, ((0, 0), (0, 0), (0, npad - n)),
                     constant_values=-30.0).reshape(bs, no, rp, 128)
        aw = jnp.pad(jnp.repeat(anchors[i, :, 0], ny * nx), (0, npad - n),
                     constant_values=1.0).reshape(rp, 128)
        ah = jnp.pad(jnp.repeat(anchors[i, :, 1], ny * nx), (0, npad - n),
                     constant_values=1.0).reshape(rp, 128)
        nmso = _nms_call(xt, aw, ah, nx, ny, n)

        hpad, wpad = _ceil(h, 8), _ceil(w, 8)
        feath = feat.transpose(0, 2, 3, 1).reshape(bs, h, w * c)
        feath = jnp.pad(feath, ((0, 0), (0, hpad - h), (0, 0)))
        a, wx = _roiy_call(nmso, feath, h, w, hpad, wpad)
        a2 = a[:, :175].reshape(bs, MDET, 7, w, c).transpose(0, 1, 3, 2, 4)
        a2 = jnp.pad(a2, ((0, 0), (0, 0), (0, wpad - w), (0, 0), (0, 0)))
        a2 = a2.reshape(bs, MDET * wpad, 7 * c)
        hh = _poolconv_call(a2, wx, cw.T, conv_b[i].reshape(1, 84), wpad, c)
        h4 = hh[:, :175].reshape(bs, MDET, 7, 7, 84).transpose(0, 1, 3, 2, 4)
        h4s.append(h4.reshape(bs * MDET, 4116))
        vms.append(nmso[:, :MDET, 4].reshape(bs * MDET))

    h4 = jnp.pad(jnp.stack(h4s), ((0, 0), (0, 4), (0, 4)))
    w1 = fc1_w.reshape(3, 100, 84, 7, 7).transpose(0, 1, 3, 4, 2)
    w1 = jnp.pad(w1.reshape(3, 100, 4116).transpose(0, 2, 1),
                 ((0, 0), (0, 4), (0, 4)))
    b1 = jnp.pad(fc1_b, ((0, 0), (0, 4))).reshape(3, 1, 104)
    w2 = jnp.pad(fc2_w.transpose(0, 2, 1), ((0, 0), (0, 4), (0, 4)))
    b2 = jnp.pad(fc2_b, ((0, 0), (0, 4))).reshape(3, 1, 104)
    w3 = jnp.pad(fc3_w.transpose(0, 2, 1), ((0, 0), (0, 4), (0, 7)))
    b3 = jnp.pad(fc3_b, ((0, 0), (0, 7))).reshape(3, 1, 8)
    vm = jnp.pad(jnp.stack(vms), ((0, 0), (0, 4))).reshape(3, 104, 1)
    vm = jnp.pad(vm, ((0, 0), (0, 0), (0, 7)))
    fco = _fc_call(h4, w1, b1, w2, b2, w3, b3, vm)
    bs = features_0.shape[0]
    pbin = fco[:, :bs * MDET, :1].reshape(3, bs, MDET, 1)
    out = jnp.zeros((3, bs, 85, 1), F32)
    return out.at[:, :, :MDET, :].set(pbin)


# speedup vs baseline: 1.3083x; 1.3083x over previous
import functools

import jax
import jax.numpy as jnp
from jax.experimental import pallas as pl
from jax.experimental.pallas import tpu as pltpu

MDET = 25
IOU_T = 0.7
CONF_T = 0.25
OFS_WH = 7680.0
HP = jax.lax.Precision.HIGHEST
F32 = jnp.float32


def _ceil(a, b):
    return -(-a // b) * b


def _red2(v, op):
    t = op(v, axis=0, keepdims=True)
    return op(t, axis=1, keepdims=True)


def _nms_body(x_ref, aw_ref, ah_ref, o_ref, s_ref, *, nx, ny, npts):
    rp = s_ref.shape[0]
    o_ref[...] = jnp.zeros_like(o_ref)
    ni = (jax.lax.broadcasted_iota(jnp.int32, (rp, 128), 0) * 128
          + jax.lax.broadcasted_iota(jnp.int32, (rp, 128), 1))
    gx = (ni % nx).astype(F32)
    gy = ((ni // nx) % ny).astype(F32)

    def sg(k):
        return jax.nn.sigmoid(x_ref[0, k])

    bx = sg(0) * 2.0 + gx - 0.5
    by = sg(1) * 2.0 + gy - 0.5
    bw = (sg(2) * 2.0) ** 2 * aw_ref[...]
    bh = (sg(3) * 2.0) ** 2 * ah_ref[...]
    x1 = bx - bw * 0.5
    y1 = by - bh * 0.5
    x2 = bx + bw * 0.5
    y2 = by + bh * 0.5
    obj = sg(4)
    m = sg(5) * obj
    ci = jnp.zeros((rp, 128), jnp.int32)
    for k in range(6, 85):
        c = sg(k) * obj
        upd = c > m
        m = jnp.where(upd, c, m)
        ci = jnp.where(upd, k - 5, ci)
    valid = (obj > CONF_T) & (m > CONF_T)
    s0 = jnp.where(valid & (ni < npts), m, -1.0)
    ofs = ci.astype(F32) * OFS_WH
    area = (x2 - x1) * (y2 - y1)
    s_ref[...] = s0
    nif = ni.astype(F32)
    lane = jax.lax.broadcasted_iota(jnp.int32, (1, 128), 1)

    def body(t, carry):
        sv = s_ref[...]
        mx = _red2(sv, jnp.max)
        sel = sv >= mx
        ii = _red2(jnp.where(sel, nif, 1e9), jnp.min)
        oh = (nif == ii).astype(F32)

        def pick(v):
            return _red2(oh * v, jnp.sum)

        px1 = pick(x1)
        py1 = pick(y1)
        px2 = pick(x2)
        py2 = pick(y2)
        pofs = pick(ofs)
        parea = pick(area)
        vflag = mx > 0.0
        qx1 = px1 + pofs
        qy1 = py1 + pofs
        qx2 = px2 + pofs
        qy2 = py2 + pofs
        xx1 = jnp.maximum(x1 + ofs, qx1)
        yy1 = jnp.maximum(y1 + ofs, qy1)
        xx2 = jnp.minimum(x2 + ofs, qx2)
        yy2 = jnp.minimum(y2 + ofs, qy2)
        inter = jnp.maximum(xx2 - xx1, 0.0) * jnp.maximum(yy2 - yy1, 0.0)
        iou = inter / (parea + area - inter + 1e-9)
        supp = (iou > IOU_T) | (oh > 0.0)
        s_ref[...] = jnp.where(supp, -1.0, sv)
        row = (jnp.where(lane == 0, jnp.where(vflag, px1, 0.0), 0.0)
               + jnp.where(lane == 1, jnp.where(vflag, py1, 0.0), 0.0)
               + jnp.where(lane == 2, jnp.where(vflag, px2, 0.0), 0.0)
               + jnp.where(lane == 3, jnp.where(vflag, py2, 0.0), 0.0)
               + jnp.where(lane == 4, jnp.where(vflag, 1.0, 0.0), 0.0))
        o_ref[0, pl.ds(t, 1), :] = row
        return carry

    jax.lax.fori_loop(0, MDET, body, 0)


def _nms_call(xt, aw, ah, nx, ny, npts):
    bs, no, rp, _ = xt.shape
    return pl.pallas_call(
        functools.partial(_nms_body, nx=nx, ny=ny, npts=npts),
        grid=(bs,),
        in_specs=[
            pl.BlockSpec((1, no, rp, 128), lambda b: (b, 0, 0, 0)),
            pl.BlockSpec((rp, 128), lambda b: (0, 0)),
            pl.BlockSpec((rp, 128), lambda b: (0, 0)),
        ],
        out_specs=pl.BlockSpec((1, 32, 128), lambda b: (b, 0, 0)),
        out_shape=jax.ShapeDtypeStruct((bs, 32, 128), F32),
        scratch_shapes=[pltpu.VMEM((rp, 128), F32)],
        compiler_params=pltpu.CompilerParams(dimension_semantics=("parallel",)),
    )(xt, aw, ah)


def _tent_w(lo, step, bound, wpad, nrows):
    col = jax.lax.broadcasted_iota(jnp.int32, (nrows, wpad), 1).astype(F32)
    pq = (jax.lax.broadcasted_iota(jnp.int32, (nrows, 1), 0) % 7).astype(F32)
    w = jnp.zeros((nrows, wpad), F32)
    for g in (0, 1):
        s = lo + step * (pq + (g + 0.5) * 0.5)
        inb = (s >= -1.0) & (s <= float(bound))
        scl = jnp.clip(s, 0.0, float(bound - 1))
        w = w + jnp.where(inb, jnp.maximum(0.0, 1.0 - jnp.abs(scl - col)), 0.0)
    return w * 0.5


def _roiy_body(nms_ref, feat_ref, a_ref, wx_ref, *, h, w, hpad, wpad):
    rsel = (jax.lax.broadcasted_iota(jnp.int32, (176, 32), 0) // 7
            == jax.lax.broadcasted_iota(jnp.int32, (176, 32), 1)).astype(F32)
    br = jax.lax.dot(rsel, nms_ref[0], precision=HP)
    x1c = br[:, 0:1]
    y1c = br[:, 1:2]
    x2c = br[:, 2:3]
    y2c = br[:, 3:4]
    bh = jnp.maximum(y2c - y1c, 1.0) / 7.0
    bw = jnp.maximum(x2c - x1c, 1.0) / 7.0
    wy = _tent_w(y1c, bh, h, hpad, 176).astype(jnp.bfloat16)
    a_ref[0] = jax.lax.dot(wy, feat_ref[0],
                           preferred_element_type=F32).astype(jnp.bfloat16)
    wx_ref[0] = _tent_w(x1c, bw, w, wpad, 176).astype(jnp.bfloat16)


def _roiy_call(nmso, feath, h, w, hpad, wpad):
    bs = feath.shape[0]
    wc = feath.shape[2]
    return pl.pallas_call(
        functools.partial(_roiy_body, h=h, w=w, hpad=hpad, wpad=wpad),
        grid=(bs,),
        in_specs=[
            pl.BlockSpec((1, 32, 128), lambda b: (b, 0, 0)),
            pl.BlockSpec((1, hpad, wc), lambda b: (b, 0, 0)),
        ],
        out_specs=[
            pl.BlockSpec((1, 176, wc), lambda b: (b, 0, 0)),
            pl.BlockSpec((1, 176, wpad), lambda b: (b, 0, 0)),
        ],
        out_shape=[
            jax.ShapeDtypeStruct((bs, 176, wc), jnp.bfloat16),
            jax.ShapeDtypeStruct((bs, 176, wpad), jnp.bfloat16),
        ],
        compiler_params=pltpu.CompilerParams(dimension_semantics=("parallel",)),
    )(nmso, feath)


def _poolconv_body(a2_ref, wx_ref, cw_ref, cb_ref, o_ref, p_scr, *, wpad, c):
    p_scr[...] = jnp.zeros_like(p_scr)
    for d in range(MDET):
        pd = jax.lax.dot(wx_ref[0, d * 7:(d + 1) * 7, :],
                         a2_ref[0, d * wpad:(d + 1) * wpad, :],
                         preferred_element_type=F32)
        p_scr[d * 7:(d + 1) * 7, :] = pd
    for py in range(7):
        hp_ = jax.lax.dot(p_scr[:, py * c:(py + 1) * c], cw_ref[...],
                          precision=HP) + cb_ref[...]
        o_ref[0, :, py * 84:(py + 1) * 84] = hp_


def _poolconv_call(a2, wx, cwt, cb, wpad, c):
    bs = a2.shape[0]
    return pl.pallas_call(
        functools.partial(_poolconv_body, wpad=wpad, c=c),
        grid=(bs,),
        in_specs=[
            pl.BlockSpec((1, MDET * wpad, 7 * c), lambda b: (b, 0, 0)),
            pl.BlockSpec((1, 176, wpad), lambda b: (b, 0, 0)),
            pl.BlockSpec((c, 84), lambda b: (0, 0)),
            pl.BlockSpec((1, 84), lambda b: (0, 0)),
        ],
        out_specs=pl.BlockSpec((1, 176, 588), lambda b: (b, 0, 0)),
        out_shape=jax.ShapeDtypeStruct((bs, 176, 588), F32),
        scratch_shapes=[pltpu.VMEM((176, 7 * c), F32)],
        compiler_params=pltpu.CompilerParams(dimension_semantics=("parallel",)),
    )(a2, wx, cwt, cb)


def _fc_body(h_ref, w1_ref, b1_ref, w2_ref, b2_ref, w3_ref, b3_ref, vm_ref,
             o_ref):
    z = jax.nn.relu(jax.lax.dot(h_ref[0], w1_ref[0], precision=HP)
                    + b1_ref[0])
    z = jax.nn.relu(jax.lax.dot(z, w2_ref[0], precision=HP) + b2_ref[0])
    z = jax.lax.dot(z, w3_ref[0], precision=HP) + b3_ref[0]
    o_ref[0] = z * vm_ref[0]


def _fc_call(h4, w1, b1, w2, b2, w3, b3, vm):
    nl = h4.shape[0]
    return pl.pallas_call(
        _fc_body,
        grid=(nl,),
        in_specs=[
            pl.BlockSpec((1, 104, 4120), lambda l: (l, 0, 0)),
            pl.BlockSpec((1, 4120, 104), lambda l: (l, 0, 0)),
            pl.BlockSpec((1, 1, 104), lambda l: (l, 0, 0)),
            pl.BlockSpec((1, 104, 104), lambda l: (l, 0, 0)),
            pl.BlockSpec((1, 1, 104), lambda l: (l, 0, 0)),
            pl.BlockSpec((1, 104, 8), lambda l: (l, 0, 0)),
            pl.BlockSpec((1, 1, 8), lambda l: (l, 0, 0)),
            pl.BlockSpec((1, 104, 8), lambda l: (l, 0, 0)),
        ],
        out_specs=pl.BlockSpec((1, 104, 8), lambda l: (l, 0, 0)),
        out_shape=jax.ShapeDtypeStruct((nl, 104, 8), F32),
        compiler_params=pltpu.CompilerParams(dimension_semantics=("parallel",)),
    )(h4, w1, b1, w2, b2, w3, b3, vm)


def kernel(features_0, features_1, features_2, x_0, x_1, x_2, conv_w0,
           conv_w1, conv_w2, conv_b, fc1_w, fc1_b, fc2_w, fc2_b, fc3_w,
           fc3_b, anchors):
    feats = (features_0, features_1, features_2)
    xs = (x_0, x_1, x_2)
    cws = (conv_w0, conv_w1, conv_w2)
    h4s, vms = [], []
    for i in range(3):
        feat, xl, cw = feats[i], xs[i], cws[i]
        bs, c, h, w = feat.shape
        _, na, ny, nx, no = xl.shape
        n = na * ny * nx
        rp = _ceil(-(-n // 128), 8)
        npad = rp * 128
        xt = xl.transpose(0, 4, 1, 2, 3).reshape(bs, no, n)
        xt = jnp.pad(xt, ((0, 0), (0, 0), (0, npad - n)),
                     constant_values=-30.0).reshape(bs, no, rp, 128)
        aw = jnp.pad(jnp.repeat(anchors[i, :, 0], ny * nx), (0, npad - n),
                     constant_values=1.0).reshape(rp, 128)
        ah = jnp.pad(jnp.repeat(anchors[i, :, 1], ny * nx), (0, npad - n),
                     constant_values=1.0).reshape(rp, 128)
        nmso = _nms_call(xt, aw, ah, nx, ny, n)

        hpad, wpad = _ceil(h, 8), _ceil(w, 8)
        feath = feat.astype(jnp.bfloat16).transpose(0, 2, 3, 1)
        feath = feath.reshape(bs, h, w * c)
        feath = jnp.pad(feath, ((0, 0), (0, hpad - h), (0, 0)))
        a, wx = _roiy_call(nmso, feath, h, w, hpad, wpad)
        a2 = a[:, :175].reshape(bs, MDET, 7, w, c).transpose(0, 1, 3, 2, 4)
        a2 = jnp.pad(a2, ((0, 0), (0, 0), (0, wpad - w), (0, 0), (0, 0)))
        a2 = a2.reshape(bs, MDET * wpad, 7 * c)
        hh = _poolconv_call(a2, wx, cw.T, conv_b[i].reshape(1, 84), wpad, c)
        h4 = hh[:, :175].reshape(bs, MDET, 7, 7, 84).transpose(0, 1, 3, 2, 4)
        h4s.append(h4.reshape(bs * MDET, 4116))
        vms.append(nmso[:, :MDET, 4].reshape(bs * MDET))

    h4 = jnp.pad(jnp.stack(h4s), ((0, 0), (0, 4), (0, 4)))
    w1 = fc1_w.reshape(3, 100, 84, 7, 7).transpose(0, 1, 3, 4, 2)
    w1 = jnp.pad(w1.reshape(3, 100, 4116).transpose(0, 2, 1),
                 ((0, 0), (0, 4), (0, 4)))
    b1 = jnp.pad(fc1_b, ((0, 0), (0, 4))).reshape(3, 1, 104)
    w2 = jnp.pad(fc2_w.transpose(0, 2, 1), ((0, 0), (0, 4), (0, 4)))
    b2 = jnp.pad(fc2_b, ((0, 0), (0, 4))).reshape(3, 1, 104)
    w3 = jnp.pad(fc3_w.transpose(0, 2, 1), ((0, 0), (0, 4), (0, 7)))
    b3 = jnp.pad(fc3_b, ((0, 0), (0, 7))).reshape(3, 1, 8)
    vm = jnp.pad(jnp.stack(vms), ((0, 0), (0, 4))).reshape(3, 104, 1)
    vm = jnp.pad(vm, ((0, 0), (0, 0), (0, 7)))
    fco = _fc_call(h4, w1, b1, w2, b2, w3, b3, vm)
    bs = features_0.shape[0]
    pbin = fco[:, :bs * MDET, :1].reshape(3, bs, MDET, 1)
    out = jnp.zeros((3, bs, 85, 1), F32)
    return out.at[:, :, :MDET, :].set(pbin)


# speedup vs baseline: 2.2021x; 1.6832x over previous
import functools

import jax
import jax.numpy as jnp
from jax.experimental import pallas as pl
from jax.experimental.pallas import tpu as pltpu

MDET = 25
IOU_T = 0.7
CONF_T = 0.25
OFS_WH = 7680.0
HP = jax.lax.Precision.HIGHEST
F32 = jnp.float32


def _ceil(a, b):
    return -(-a // b) * b


def _red2(v, op):
    t = op(v, axis=0, keepdims=True)
    return op(t, axis=1, keepdims=True)


def _nms_body(x_ref, aw_ref, ah_ref, o_ref, s_ref, *, nx, ny, npts):
    rp = s_ref.shape[0]
    o_ref[...] = jnp.zeros_like(o_ref)
    ni = (jax.lax.broadcasted_iota(jnp.int32, (rp, 128), 0) * 128
          + jax.lax.broadcasted_iota(jnp.int32, (rp, 128), 1))
    gx = (ni % nx).astype(F32)
    gy = ((ni // nx) % ny).astype(F32)

    def sg(k):
        return jax.nn.sigmoid(x_ref[0, k])

    bx = sg(0) * 2.0 + gx - 0.5
    by = sg(1) * 2.0 + gy - 0.5
    bw = (sg(2) * 2.0) ** 2 * aw_ref[...]
    bh = (sg(3) * 2.0) ** 2 * ah_ref[...]
    x1 = bx - bw * 0.5
    y1 = by - bh * 0.5
    x2 = bx + bw * 0.5
    y2 = by + bh * 0.5
    obj = sg(4)
    m = sg(5) * obj
    ci = jnp.zeros((rp, 128), jnp.int32)
    for k in range(6, 85):
        c = sg(k) * obj
        upd = c > m
        m = jnp.where(upd, c, m)
        ci = jnp.where(upd, k - 5, ci)
    valid = (obj > CONF_T) & (m > CONF_T)
    s0 = jnp.where(valid & (ni < npts), m, -1.0)
    ofs = ci.astype(F32) * OFS_WH
    area = (x2 - x1) * (y2 - y1)
    s_ref[...] = s0
    nif = ni.astype(F32)
    lane = jax.lax.broadcasted_iota(jnp.int32, (1, 128), 1)

    def body(t, carry):
        sv = s_ref[...]
        mx = _red2(sv, jnp.max)
        sel = sv >= mx
        ii = _red2(jnp.where(sel, nif, 1e9), jnp.min)
        oh = (nif == ii).astype(F32)

        def pick(v):
            return _red2(oh * v, jnp.sum)

        px1 = pick(x1)
        py1 = pick(y1)
        px2 = pick(x2)
        py2 = pick(y2)
        pofs = pick(ofs)
        parea = pick(area)
        vflag = mx > 0.0
        qx1 = px1 + pofs
        qy1 = py1 + pofs
        qx2 = px2 + pofs
        qy2 = py2 + pofs
        xx1 = jnp.maximum(x1 + ofs, qx1)
        yy1 = jnp.maximum(y1 + ofs, qy1)
        xx2 = jnp.minimum(x2 + ofs, qx2)
        yy2 = jnp.minimum(y2 + ofs, qy2)
        inter = jnp.maximum(xx2 - xx1, 0.0) * jnp.maximum(yy2 - yy1, 0.0)
        iou = inter / (parea + area - inter + 1e-9)
        supp = (iou > IOU_T) | (oh > 0.0)
        s_ref[...] = jnp.where(supp, -1.0, sv)
        row = (jnp.where(lane == 0, jnp.where(vflag, px1, 0.0), 0.0)
               + jnp.where(lane == 1, jnp.where(vflag, py1, 0.0), 0.0)
               + jnp.where(lane == 2, jnp.where(vflag, px2, 0.0), 0.0)
               + jnp.where(lane == 3, jnp.where(vflag, py2, 0.0), 0.0)
               + jnp.where(lane == 4, jnp.where(vflag, 1.0, 0.0), 0.0))
        o_ref[0, pl.ds(t, 1), :] = row
        return carry

    jax.lax.fori_loop(0, MDET, body, 0)


def _nms_call(xt, aw, ah, nx, ny, npts):
    bs, no, rp, _ = xt.shape
    return pl.pallas_call(
        functools.partial(_nms_body, nx=nx, ny=ny, npts=npts),
        grid=(bs,),
        in_specs=[
            pl.BlockSpec((1, no, rp, 128), lambda b: (b, 0, 0, 0)),
            pl.BlockSpec((rp, 128), lambda b: (0, 0)),
            pl.BlockSpec((rp, 128), lambda b: (0, 0)),
        ],
        out_specs=pl.BlockSpec((1, 32, 128), lambda b: (b, 0, 0)),
        out_shape=jax.ShapeDtypeStruct((bs, 32, 128), F32),
        scratch_shapes=[pltpu.VMEM((rp, 128), F32)],
        compiler_params=pltpu.CompilerParams(dimension_semantics=("parallel",)),
    )(xt, aw, ah)


def _tent_w(lo, step, bound, wpad, nrows):
    col = jax.lax.broadcasted_iota(jnp.int32, (nrows, wpad), 1).astype(F32)
    pq = (jax.lax.broadcasted_iota(jnp.int32, (nrows, 1), 0) % 7).astype(F32)
    w = jnp.zeros((nrows, wpad), F32)
    for g in (0, 1):
        s = lo + step * (pq + (g + 0.5) * 0.5)
        inb = (s >= -1.0) & (s <= float(bound))
        scl = jnp.clip(s, 0.0, float(bound - 1))
        w = w + jnp.where(inb, jnp.maximum(0.0, 1.0 - jnp.abs(scl - col)), 0.0)
    return w * 0.5


def _roiy_body(nms_ref, feat_ref, a_ref, wx_ref, *, h, w, hpad, wpad):
    rsel = (jax.lax.broadcasted_iota(jnp.int32, (176, 32), 0) // 7
            == jax.lax.broadcasted_iota(jnp.int32, (176, 32), 1)).astype(F32)
    br = jax.lax.dot(rsel, nms_ref[0], precision=HP)
    x1c = br[:, 0:1]
    y1c = br[:, 1:2]
    x2c = br[:, 2:3]
    y2c = br[:, 3:4]
    bh = jnp.maximum(y2c - y1c, 1.0) / 7.0
    bw = jnp.maximum(x2c - x1c, 1.0) / 7.0
    wy = _tent_w(y1c, bh, h, hpad, 176).astype(jnp.bfloat16)
    a_ref[0] = jax.lax.dot(wy, feat_ref[0],
                           preferred_element_type=F32).astype(jnp.bfloat16)
    wx_ref[0] = _tent_w(x1c, bw, w, wpad, 176).astype(jnp.bfloat16)


def _roiy_call(nmso, feath, h, w, hpad, wpad):
    bs = feath.shape[0]
    wc = feath.shape[2]
    return pl.pallas_call(
        functools.partial(_roiy_body, h=h, w=w, hpad=hpad, wpad=wpad),
        grid=(bs,),
        in_specs=[
            pl.BlockSpec((1, 32, 128), lambda b: (b, 0, 0)),
            pl.BlockSpec((1, hpad, wc), lambda b: (b, 0, 0)),
        ],
        out_specs=[
            pl.BlockSpec((1, 176, wc), lambda b: (b, 0, 0)),
            pl.BlockSpec((1, 176, wpad), lambda b: (b, 0, 0)),
        ],
        out_shape=[
            jax.ShapeDtypeStruct((bs, 176, wc), jnp.bfloat16),
            jax.ShapeDtypeStruct((bs, 176, wpad), jnp.bfloat16),
        ],
        compiler_params=pltpu.CompilerParams(dimension_semantics=("parallel",)),
    )(nmso, feath)


def _poolconv_body(a2_ref, wx_ref, cw_ref, cb_ref, o_ref, p_scr, *, wpad, c):
    w = a2_ref.shape[2] // c
    p_scr[...] = jnp.zeros_like(p_scr)
    for d in range(MDET):
        ad = a2_ref[0, d * 7:(d + 1) * 7, :]
        a2d = ad.reshape(7, w, c).transpose(1, 0, 2).reshape(w, 7 * c)
        pd = jax.lax.dot(wx_ref[0, d * 7:(d + 1) * 7, :w], a2d,
                         preferred_element_type=F32)
        p_scr[d * 7:(d + 1) * 7, :] = pd
    for py in range(7):
        hp_ = jax.lax.dot(p_scr[:, py * c:(py + 1) * c].astype(jnp.bfloat16),
                          cw_ref[...], preferred_element_type=F32) + cb_ref[...]
        o_ref[0, :, py * 84:(py + 1) * 84] = hp_


def _poolconv_call(a2, wx, cwt, cb, wpad, c):
    bs = a2.shape[0]
    return pl.pallas_call(
        functools.partial(_poolconv_body, wpad=wpad, c=c),
        grid=(bs,),
        in_specs=[
            pl.BlockSpec((1, 176, a2.shape[2]), lambda b: (b, 0, 0)),
            pl.BlockSpec((1, 176, wpad), lambda b: (b, 0, 0)),
            pl.BlockSpec((c, 84), lambda b: (0, 0)),
            pl.BlockSpec((1, 84), lambda b: (0, 0)),
        ],
        out_specs=pl.BlockSpec((1, 176, 588), lambda b: (b, 0, 0)),
        out_shape=jax.ShapeDtypeStruct((bs, 176, 588), F32),
        scratch_shapes=[pltpu.VMEM((176, 7 * c), F32)],
        compiler_params=pltpu.CompilerParams(dimension_semantics=("parallel",)),
    )(a2, wx, cwt, cb)


def _fc_body(h_ref, w1_ref, b1_ref, w2_ref, b2_ref, w3_ref, b3_ref, vm_ref,
             o_ref):
    z = jax.nn.relu(jax.lax.dot(h_ref[0], w1_ref[0], precision=HP)
                    + b1_ref[0])
    z = jax.nn.relu(jax.lax.dot(z, w2_ref[0], precision=HP) + b2_ref[0])
    z = jax.lax.dot(z, w3_ref[0], precision=HP) + b3_ref[0]
    o_ref[0] = z * vm_ref[0]


def _fc_call(h4, w1, b1, w2, b2, w3, b3, vm):
    nl = h4.shape[0]
    return pl.pallas_call(
        _fc_body,
        grid=(nl,),
        in_specs=[
            pl.BlockSpec((1, 104, 4120), lambda l: (l, 0, 0)),
            pl.BlockSpec((1, 4120, 104), lambda l: (l, 0, 0)),
            pl.BlockSpec((1, 1, 104), lambda l: (l, 0, 0)),
            pl.BlockSpec((1, 104, 104), lambda l: (l, 0, 0)),
            pl.BlockSpec((1, 1, 104), lambda l: (l, 0, 0)),
            pl.BlockSpec((1, 104, 8), lambda l: (l, 0, 0)),
            pl.BlockSpec((1, 1, 8), lambda l: (l, 0, 0)),
            pl.BlockSpec((1, 104, 8), lambda l: (l, 0, 0)),
        ],
        out_specs=pl.BlockSpec((1, 104, 8), lambda l: (l, 0, 0)),
        out_shape=jax.ShapeDtypeStruct((nl, 104, 8), F32),
        compiler_params=pltpu.CompilerParams(dimension_semantics=("parallel",)),
    )(h4, w1, b1, w2, b2, w3, b3, vm)


def kernel(features_0, features_1, features_2, x_0, x_1, x_2, conv_w0,
           conv_w1, conv_w2, conv_b, fc1_w, fc1_b, fc2_w, fc2_b, fc3_w,
           fc3_b, anchors):
    feats = (features_0, features_1, features_2)
    xs = (x_0, x_1, x_2)
    cws = (conv_w0, conv_w1, conv_w2)
    h4s, vms = [], []
    for i in range(3):
        feat, xl, cw = feats[i], xs[i], cws[i]
        bs, c, h, w = feat.shape
        _, na, ny, nx, no = xl.shape
        n = na * ny * nx
        rp = _ceil(-(-n // 128), 8)
        npad = rp * 128
        xt = xl.transpose(0, 4, 1, 2, 3).reshape(bs, no, n)
        xt = jnp.pad(xt, ((0, 0), (0, 0), (0, npad - n)),
                     constant_values=-30.0).reshape(bs, no, rp, 128)
        aw = jnp.pad(jnp.repeat(anchors[i, :, 0], ny * nx), (0, npad - n),
                     constant_values=1.0).reshape(rp, 128)
        ah = jnp.pad(jnp.repeat(anchors[i, :, 1], ny * nx), (0, npad - n),
                     constant_values=1.0).reshape(rp, 128)
        nmso = _nms_call(xt, aw, ah, nx, ny, n)

        hpad, wpad = _ceil(h, 8), _ceil(w, 8)
        feath = feat.astype(jnp.bfloat16).transpose(0, 2, 3, 1)
        feath = feath.reshape(bs, h, w * c)
        feath = jnp.pad(feath, ((0, 0), (0, hpad - h), (0, 0)))
        a, wx = _roiy_call(nmso, feath, h, w, hpad, wpad)
        hh = _poolconv_call(a, wx, cw.T.astype(jnp.bfloat16),
                            conv_b[i].reshape(1, 84), wpad, c)
        h4 = hh[:, :175].reshape(bs, MDET, 7, 7, 84).transpose(0, 1, 3, 2, 4)
        h4s.append(h4.reshape(bs * MDET, 4116))
        vms.append(nmso[:, :MDET, 4].reshape(bs * MDET))

    h4 = jnp.pad(jnp.stack(h4s), ((0, 0), (0, 4), (0, 4)))
    w1 = fc1_w.reshape(3, 100, 84, 7, 7).transpose(0, 1, 3, 4, 2)
    w1 = jnp.pad(w1.reshape(3, 100, 4116).transpose(0, 2, 1),
                 ((0, 0), (0, 4), (0, 4)))
    b1 = jnp.pad(fc1_b, ((0, 0), (0, 4))).reshape(3, 1, 104)
    w2 = jnp.pad(fc2_w.transpose(0, 2, 1), ((0, 0), (0, 4), (0, 4)))
    b2 = jnp.pad(fc2_b, ((0, 0), (0, 4))).reshape(3, 1, 104)
    w3 = jnp.pad(fc3_w.transpose(0, 2, 1), ((0, 0), (0, 4), (0, 7)))
    b3 = jnp.pad(fc3_b, ((0, 0), (0, 7))).reshape(3, 1, 8)
    vm = jnp.pad(jnp.stack(vms), ((0, 0), (0, 4))).reshape(3, 104, 1)
    vm = jnp.pad(vm, ((0, 0), (0, 0), (0, 7)))
    fco = _fc_call(h4, w1, b1, w2, b2, w3, b3, vm)
    bs = features_0.shape[0]
    pbin = fco[:, :bs * MDET, :1].reshape(3, bs, MDET, 1)
    out = jnp.zeros((3, bs, 85, 1), F32)
    return out.at[:, :, :MDET, :].set(pbin)


# speedup vs baseline: 2.3344x; 1.0600x over previous
import functools

import jax
import jax.numpy as jnp
from jax.experimental import pallas as pl
from jax.experimental.pallas import tpu as pltpu

MDET = 25
IOU_T = 0.7
CONF_T = 0.25
OFS_WH = 7680.0
HP = jax.lax.Precision.HIGHEST
F32 = jnp.float32


def _ceil(a, b):
    return -(-a // b) * b


def _red2(v, op):
    t = op(v, axis=0, keepdims=True)
    return op(t, axis=1, keepdims=True)


def _nms_body(x_ref, aw_ref, ah_ref, o_ref, s_ref, *, nx, ny, npts):
    rp = s_ref.shape[0]
    o_ref[...] = jnp.zeros_like(o_ref)
    ni = (jax.lax.broadcasted_iota(jnp.int32, (rp, 128), 0) * 128
          + jax.lax.broadcasted_iota(jnp.int32, (rp, 128), 1))
    gx = (ni % nx).astype(F32)
    gy = ((ni // nx) % ny).astype(F32)

    def sg(k):
        return jax.nn.sigmoid(x_ref[0, k])

    bx = sg(0) * 2.0 + gx - 0.5
    by = sg(1) * 2.0 + gy - 0.5
    bw = (sg(2) * 2.0) ** 2 * aw_ref[...]
    bh = (sg(3) * 2.0) ** 2 * ah_ref[...]
    x1 = bx - bw * 0.5
    y1 = by - bh * 0.5
    x2 = bx + bw * 0.5
    y2 = by + bh * 0.5
    obj = sg(4)
    m = sg(5) * obj
    ci = jnp.zeros((rp, 128), jnp.int32)
    for k in range(6, 85):
        c = sg(k) * obj
        upd = c > m
        m = jnp.where(upd, c, m)
        ci = jnp.where(upd, k - 5, ci)
    valid = (obj > CONF_T) & (m > CONF_T)
    s0 = jnp.where(valid & (ni < npts), m, -1.0)
    ofs = ci.astype(F32) * OFS_WH
    area = (x2 - x1) * (y2 - y1)
    s_ref[...] = s0
    nif = ni.astype(F32)
    lane = jax.lax.broadcasted_iota(jnp.int32, (1, 128), 1)

    def body(t, carry):
        sv = s_ref[...]
        mx = _red2(sv, jnp.max)
        sel = sv >= mx
        ii = _red2(jnp.where(sel, nif, 1e9), jnp.min)
        oh = (nif == ii).astype(F32)

        def pick(v):
            return _red2(oh * v, jnp.sum)

        px1 = pick(x1)
        py1 = pick(y1)
        px2 = pick(x2)
        py2 = pick(y2)
        pofs = pick(ofs)
        parea = pick(area)
        vflag = mx > 0.0
        qx1 = px1 + pofs
        qy1 = py1 + pofs
        qx2 = px2 + pofs
        qy2 = py2 + pofs
        xx1 = jnp.maximum(x1 + ofs, qx1)
        yy1 = jnp.maximum(y1 + ofs, qy1)
        xx2 = jnp.minimum(x2 + ofs, qx2)
        yy2 = jnp.minimum(y2 + ofs, qy2)
        inter = jnp.maximum(xx2 - xx1, 0.0) * jnp.maximum(yy2 - yy1, 0.0)
        iou = inter / (parea + area - inter + 1e-9)
        supp = (iou > IOU_T) | (oh > 0.0)
        s_ref[...] = jnp.where(supp, -1.0, sv)
        row = (jnp.where(lane == 0, jnp.where(vflag, px1, 0.0), 0.0)
               + jnp.where(lane == 1, jnp.where(vflag, py1, 0.0), 0.0)
               + jnp.where(lane == 2, jnp.where(vflag, px2, 0.0), 0.0)
               + jnp.where(lane == 3, jnp.where(vflag, py2, 0.0), 0.0)
               + jnp.where(lane == 4, jnp.where(vflag, 1.0, 0.0), 0.0))
        o_ref[0, pl.ds(t, 1), :] = row
        return carry

    jax.lax.fori_loop(0, MDET, body, 0)


def _nms_call(xt, aw, ah, nx, ny, npts):
    bs, no, rp, _ = xt.shape
    return pl.pallas_call(
        functools.partial(_nms_body, nx=nx, ny=ny, npts=npts),
        grid=(bs,),
        in_specs=[
            pl.BlockSpec((1, no, rp, 128), lambda b: (b, 0, 0, 0)),
            pl.BlockSpec((rp, 128), lambda b: (0, 0)),
            pl.BlockSpec((rp, 128), lambda b: (0, 0)),
        ],
        out_specs=pl.BlockSpec((1, 32, 128), lambda b: (b, 0, 0)),
        out_shape=jax.ShapeDtypeStruct((bs, 32, 128), F32),
        scratch_shapes=[pltpu.VMEM((rp, 128), F32)],
        compiler_params=pltpu.CompilerParams(dimension_semantics=("parallel",)),
    )(xt, aw, ah)


def _tent_w(lo, step, bound, wpad, nrows):
    col = jax.lax.broadcasted_iota(jnp.int32, (nrows, wpad), 1).astype(F32)
    pq = (jax.lax.broadcasted_iota(jnp.int32, (nrows, 1), 0) % 7).astype(F32)
    w = jnp.zeros((nrows, wpad), F32)
    for g in (0, 1):
        s = lo + step * (pq + (g + 0.5) * 0.5)
        inb = (s >= -1.0) & (s <= float(bound))
        scl = jnp.clip(s, 0.0, float(bound - 1))
        w = w + jnp.where(inb, jnp.maximum(0.0, 1.0 - jnp.abs(scl - col)), 0.0)
    return w * 0.5


def _roi_body(nms_ref, feat_ref, cw_ref, cb_ref, o_ref, a_scr, p_scr,
              *, h, w, hpad, wpad, c):
    rsel = (jax.lax.broadcasted_iota(jnp.int32, (176, 32), 0) // 7
            == jax.lax.broadcasted_iota(jnp.int32, (176, 32), 1)).astype(F32)
    br = jax.lax.dot(rsel, nms_ref[0], precision=HP)
    x1c = br[:, 0:1]
    y1c = br[:, 1:2]
    x2c = br[:, 2:3]
    y2c = br[:, 3:4]
    bh = jnp.maximum(y2c - y1c, 1.0) / 7.0
    bw = jnp.maximum(x2c - x1c, 1.0) / 7.0
    wy = _tent_w(y1c, bh, h, hpad, 176).astype(jnp.bfloat16)
    a_scr[...] = jax.lax.dot(wy, feat_ref[0],
                             preferred_element_type=F32).astype(jnp.bfloat16)
    wx = _tent_w(x1c, bw, w, wpad, 176).astype(jnp.bfloat16)
    p_scr[...] = jnp.zeros_like(p_scr)
    for d in range(MDET):
        ad = a_scr[d * 7:(d + 1) * 7, :]
        a2d = ad.reshape(7, w, c).transpose(1, 0, 2).reshape(w, 7 * c)
        pd = jax.lax.dot(wx[d * 7:(d + 1) * 7, :w], a2d,
                         preferred_element_type=F32)
        p_scr[d * 7:(d + 1) * 7, :] = pd
    for py in range(7):
        hp_ = jax.lax.dot(p_scr[:, py * c:(py + 1) * c].astype(jnp.bfloat16),
                          cw_ref[...], preferred_element_type=F32) + cb_ref[...]
        o_ref[0, :, py * 84:(py + 1) * 84] = hp_


def _roi_call(nmso, feath, cwt, cb, h, w, hpad, wpad, c):
    bs = feath.shape[0]
    wc = feath.shape[2]
    return pl.pallas_call(
        functools.partial(_roi_body, h=h, w=w, hpad=hpad, wpad=wpad, c=c),
        grid=(bs,),
        in_specs=[
            pl.BlockSpec((1, 32, 128), lambda b: (b, 0, 0)),
            pl.BlockSpec((1, hpad, wc), lambda b: (b, 0, 0)),
            pl.BlockSpec((c, 84), lambda b: (0, 0)),
            pl.BlockSpec((1, 84), lambda b: (0, 0)),
        ],
        out_specs=pl.BlockSpec((1, 176, 588), lambda b: (b, 0, 0)),
        out_shape=jax.ShapeDtypeStruct((bs, 176, 588), F32),
        scratch_shapes=[pltpu.VMEM((176, wc), jnp.bfloat16),
                        pltpu.VMEM((176, 7 * c), F32)],
        compiler_params=pltpu.CompilerParams(dimension_semantics=("parallel",)),
    )(nmso, feath, cwt, cb)


def _fc_body(h_ref, w1_ref, b1_ref, w2_ref, b2_ref, w3_ref, b3_ref, vm_ref,
             o_ref):
    z = jax.nn.relu(jax.lax.dot(h_ref[0], w1_ref[0], precision=HP)
                    + b1_ref[0])
    z = jax.nn.relu(jax.lax.dot(z, w2_ref[0], precision=HP) + b2_ref[0])
    z = jax.lax.dot(z, w3_ref[0], precision=HP) + b3_ref[0]
    o_ref[0] = z * vm_ref[0]


def _fc_call(h4, w1, b1, w2, b2, w3, b3, vm):
    nl = h4.shape[0]
    return pl.pallas_call(
        _fc_body,
        grid=(nl,),
        in_specs=[
            pl.BlockSpec((1, 104, 4120), lambda l: (l, 0, 0)),
            pl.BlockSpec((1, 4120, 104), lambda l: (l, 0, 0)),
            pl.BlockSpec((1, 1, 104), lambda l: (l, 0, 0)),
            pl.BlockSpec((1, 104, 104), lambda l: (l, 0, 0)),
            pl.BlockSpec((1, 1, 104), lambda l: (l, 0, 0)),
            pl.BlockSpec((1, 104, 8), lambda l: (l, 0, 0)),
            pl.BlockSpec((1, 1, 8), lambda l: (l, 0, 0)),
            pl.BlockSpec((1, 104, 8), lambda l: (l, 0, 0)),
        ],
        out_specs=pl.BlockSpec((1, 104, 8), lambda l: (l, 0, 0)),
        out_shape=jax.ShapeDtypeStruct((nl, 104, 8), F32),
        compiler_params=pltpu.CompilerParams(dimension_semantics=("parallel",)),
    )(h4, w1, b1, w2, b2, w3, b3, vm)


def kernel(features_0, features_1, features_2, x_0, x_1, x_2, conv_w0,
           conv_w1, conv_w2, conv_b, fc1_w, fc1_b, fc2_w, fc2_b, fc3_w,
           fc3_b, anchors):
    feats = (features_0, features_1, features_2)
    xs = (x_0, x_1, x_2)
    cws = (conv_w0, conv_w1, conv_w2)
    h4s, vms = [], []
    for i in range(3):
        feat, xl, cw = feats[i], xs[i], cws[i]
        bs, c, h, w = feat.shape
        _, na, ny, nx, no = xl.shape
        n = na * ny * nx
        rp = _ceil(-(-n // 128), 8)
        npad = rp * 128
        xt = xl.transpose(0, 4, 1, 2, 3).reshape(bs, no, n)
        xt = jnp.pad(xt, ((0, 0), (0, 0), (0, npad - n)),
                     constant_values=-30.0).reshape(bs, no, rp, 128)
        aw = jnp.pad(jnp.repeat(anchors[i, :, 0], ny * nx), (0, npad - n),
                     constant_values=1.0).reshape(rp, 128)
        ah = jnp.pad(jnp.repeat(anchors[i, :, 1], ny * nx), (0, npad - n),
                     constant_values=1.0).reshape(rp, 128)
        nmso = _nms_call(xt, aw, ah, nx, ny, n)

        hpad, wpad = _ceil(h, 8), _ceil(w, 8)
        feath = feat.astype(jnp.bfloat16).transpose(0, 2, 3, 1)
        feath = feath.reshape(bs, h, w * c)
        feath = jnp.pad(feath, ((0, 0), (0, hpad - h), (0, 0)))
        hh = _roi_call(nmso, feath, cw.T.astype(jnp.bfloat16),
                       conv_b[i].reshape(1, 84), h, w, hpad, wpad, c)
        h4 = hh[:, :175].reshape(bs, MDET, 7, 7, 84).transpose(0, 1, 3, 2, 4)
        h4s.append(h4.reshape(bs * MDET, 4116))
        vms.append(nmso[:, :MDET, 4].reshape(bs * MDET))

    h4 = jnp.pad(jnp.stack(h4s), ((0, 0), (0, 4), (0, 4)))
    w1 = fc1_w.reshape(3, 100, 84, 7, 7).transpose(0, 1, 3, 4, 2)
    w1 = jnp.pad(w1.reshape(3, 100, 4116).transpose(0, 2, 1),
                 ((0, 0), (0, 4), (0, 4)))
    b1 = jnp.pad(fc1_b, ((0, 0), (0, 4))).reshape(3, 1, 104)
    w2 = jnp.pad(fc2_w.transpose(0, 2, 1), ((0, 0), (0, 4), (0, 4)))
    b2 = jnp.pad(fc2_b, ((0, 0), (0, 4))).reshape(3, 1, 104)
    w3 = jnp.pad(fc3_w.transpose(0, 2, 1), ((0, 0), (0, 4), (0, 7)))
    b3 = jnp.pad(fc3_b, ((0, 0), (0, 7))).reshape(3, 1, 8)
    vm = jnp.pad(jnp.stack(vms), ((0, 0), (0, 4))).reshape(3, 104, 1)
    vm = jnp.pad(vm, ((0, 0), (0, 0), (0, 7)))
    fco = _fc_call(h4, w1, b1, w2, b2, w3, b3, vm)
    bs = features_0.shape[0]
    pbin = fco[:, :bs * MDET, :1].reshape(3, bs, MDET, 1)
    out = jnp.zeros((3, bs, 85, 1), F32)
    return out.at[:, :, :MDET, :].set(pbin)
